# baseline (speedup 1.0000x reference)
import jax
import jax.numpy as jnp
from jax import lax
from jax.experimental import pallas as pl
from jax.experimental.pallas import tpu as pltpu
from jax.experimental.pallas import tpu_sc as plsc

F32 = jnp.float32

NC = 2
NS = 16
NW = NC * NS

CH = 125
DEGW = 16
ZR = 64
NBUF = 4


def _agg_kernel(n_pad, fh, n_ch, with_deg):
    rows_per_tile = n_pad // NS
    n_zb = rows_per_tile // ZR

    mesh = plsc.VectorSubcoreMesh(
        core_axis_name="c", subcore_axis_name="s",
        num_cores=NC, num_subcores=NS)

    out_type = [jax.ShapeDtypeStruct((NC, n_pad, fh), F32),
                jax.ShapeDtypeStruct((NC, n_pad, fh), F32)]
    scratch = (
        [pltpu.VMEM((n_ch, CH), jnp.int32),
         pltpu.VMEM((n_ch, CH), jnp.int32)]
        + [pltpu.VMEM((CH, fh), F32)] * NBUF
        + [pltpu.VMEM((ZR, fh), F32),
           pltpu.VMEM((ZR, fh), F32),
           pltpu.VMEM_SHARED((n_pad, fh), F32)]
        + [pltpu.SemaphoreType.DMA] * NBUF
    )
    if with_deg:
        out_type.append(jax.ShapeDtypeStruct((NC, n_pad, DEGW), F32))
        scratch += [
            pltpu.VMEM((CH, DEGW), F32),
            pltpu.VMEM((rows_per_tile, DEGW), F32),
            pltpu.VMEM_SHARED((n_pad, DEGW), F32),
        ]

    def body(ya_hbm, yb_hbm, src_hbm, dst_hbm, outa_hbm, outb_hbm, *rest):
        if with_deg:
            deg_hbm = rest[0]
            rest = rest[1:]
        srcv, dstv = rest[0], rest[1]
        rows = rest[2:2 + NBUF]
        zb, stg, acc_sh = rest[2 + NBUF:5 + NBUF]
        sems = rest[5 + NBUF:5 + 2 * NBUF]
        if with_deg:
            ones_v, degz, deg_sh = rest[5 + 2 * NBUF:]
        c = lax.axis_index("c")
        s = lax.axis_index("s")
        wid = c * NS + s

        def zrow(r, _):
            for k in range(fh // 16):
                zb[r, pl.ds(k * 16, 16)] = jnp.zeros((16,), F32)
            return 0
        lax.fori_loop(0, ZR, zrow, 0)
        for j in range(n_zb):
            pltpu.sync_copy(zb, acc_sh.at[pl.ds(s * rows_per_tile + j * ZR, ZR)])
        if with_deg:
            def zdrow(r, _):
                degz[r] = jnp.zeros((DEGW,), F32)
                return 0
            lax.fori_loop(0, rows_per_tile, zdrow, 0)
            pltpu.sync_copy(degz, deg_sh.at[pl.ds(s * rows_per_tile, rows_per_tile)])

            def orow(r, _):
                ones_v[r] = jnp.ones((DEGW,), F32)
                return 0
            lax.fori_loop(0, CH, orow, 0)

        pltpu.sync_copy(src_hbm.at[wid], srcv)
        pltpu.sync_copy(dst_hbm.at[wid], dstv)

        plsc.subcore_barrier()

        for y_hbm, out_hbm, first in ((ya_hbm, outa_hbm, True),
                                      (yb_hbm, outb_hbm, False)):
            count_deg = with_deg and first
            for b in range(NBUF):
                pltpu.async_copy(y_hbm.at[srcv.at[b]], rows[b], sems[b])

            def step(i, _):
                j0 = NBUF * i
                for b in range(NBUF):
                    j = j0 + b
                    pltpu.make_async_copy(
                        y_hbm.at[srcv.at[j]], rows[b], sems[b]).wait()
                    pltpu.sync_copy(rows[b], acc_sh.at[dstv.at[j]],
                                    add=True)
                    if count_deg:
                        pltpu.sync_copy(ones_v, deg_sh.at[dstv.at[j]],
                                        add=True)

                    @pl.when(j + NBUF < n_ch)
                    def _(b=b, j=j):
                        pltpu.async_copy(
                            y_hbm.at[srcv.at[j + NBUF]], rows[b], sems[b])
                return 0
            lax.fori_loop(0, n_ch // NBUF, step, 0)

            plsc.subcore_barrier()

            for j in range(n_zb):
                r0 = s * rows_per_tile + j * ZR
                pltpu.sync_copy(acc_sh.at[pl.ds(r0, ZR)], stg)
                pltpu.sync_copy(stg, out_hbm.at[c].at[pl.ds(r0, ZR)])
                if first:
                    pltpu.sync_copy(zb, acc_sh.at[pl.ds(r0, ZR)])
            if count_deg:
                r0 = s * rows_per_tile
                pltpu.sync_copy(deg_sh.at[pl.ds(r0, rows_per_tile)], degz)
                pltpu.sync_copy(degz, deg_hbm.at[c].at[pl.ds(r0, rows_per_tile)])

            if first:
                plsc.subcore_barrier()

    return pl.kernel(body, out_type=tuple(out_type), mesh=mesh,
                     scratch_types=list(scratch),
                     compiler_params=pltpu.CompilerParams(
                         use_tc_tiling_on_sc=False))


def _mm1_body(x_ref, w_ref, oa_ref, ob_ref):
    fh = oa_ref.shape[1]
    y = jnp.dot(x_ref[...], w_ref[...], preferred_element_type=F32)
    oa_ref[...] = y[:, :fh]
    ob_ref[...] = y[:, fh:]


def _mean(pa, pb, d):
    deg = d[0][:, :1] + d[1][:, :1]
    rd = 1.0 / jnp.maximum(deg, 1.0)
    return jnp.concatenate([(pa[0] + pa[1]) * rd, (pb[0] + pb[1]) * rd],
                           axis=1)


def _mid_body(x_ref, pa_ref, pb_ref, d_ref, ws_ref, wn_ref, b_ref,
              h_ref, ya_ref, yb_ref):
    fh = ya_ref.shape[1]
    mean = _mean(pa_ref[...], pb_ref[...], d_ref[...])
    h = jnp.maximum(
        jnp.dot(x_ref[...], ws_ref[...], preferred_element_type=F32)
        + mean + b_ref[...], 0.0)
    h_ref[...] = h
    y = jnp.dot(h, wn_ref[...], preferred_element_type=F32)
    ya_ref[...] = y[:, :fh]
    yb_ref[...] = y[:, fh:]


def _fin_body(h_ref, pa_ref, pb_ref, d_ref, ws_ref, b_ref,
              wfc_ref, bfc_ref, o_ref):
    mean = _mean(pa_ref[...], pb_ref[...], d_ref[...])
    h2 = (jnp.dot(h_ref[...], ws_ref[...], preferred_element_type=F32)
          + mean + b_ref[...])
    o_ref[...] = (jnp.dot(h2, wfc_ref[...], preferred_element_type=F32)
                  + bfc_ref[...])


def kernel(x, edge_index, W_self1, W_neigh1, b1, W_self2, W_neigh2, b2,
           Wfc, bfc):
    n_nodes, feat = x.shape
    n_edges = edge_index.shape[1]
    nclass = Wfc.shape[1]
    fh = feat // 2
    n_pad = -(-n_nodes // (NS * 128)) * (NS * 128)

    e_per_tile = n_edges // NW
    n_ch = e_per_tile // CH
    src3d = edge_index[0].reshape(NW, n_ch, CH)
    dst3d = edge_index[1].reshape(NW, n_ch, CH)

    agg_deg = _agg_kernel(n_pad, fh, n_ch, with_deg=True)
    agg = _agg_kernel(n_pad, fh, n_ch, with_deg=False)

    BR = 1000
    grid = (n_nodes // BR,)
    row_blk = pl.BlockSpec((BR, feat), lambda i: (i, 0))
    half_blk = pl.BlockSpec((BR, fh), lambda i: (i, 0))
    part_blk = pl.BlockSpec((NC, BR, fh), lambda i: (0, i, 0))
    deg_blk = pl.BlockSpec((NC, BR, DEGW), lambda i: (0, i, 0))
    w_blk = pl.BlockSpec((feat, feat), lambda i: (0, 0))
    b_blk = pl.BlockSpec((1, feat), lambda i: (0, 0))
    half_shape = jax.ShapeDtypeStruct((n_nodes, fh), F32)

    y1a, y1b = pl.pallas_call(
        _mm1_body, grid=grid,
        in_specs=[row_blk, w_blk],
        out_specs=(half_blk, half_blk),
        out_shape=(half_shape, half_shape),
    )(x, W_neigh1)

    pa, pb, degp = agg_deg(y1a, y1b, src3d, dst3d)

    h1, y2a, y2b = pl.pallas_call(
        _mid_body, grid=grid,
        in_specs=[row_blk, part_blk, part_blk, deg_blk,
                  w_blk, w_blk, b_blk],
        out_specs=(row_blk, half_blk, half_blk),
        out_shape=(jax.ShapeDtypeStruct((n_nodes, feat), F32),
                   half_shape, half_shape),
    )(x, pa, pb, degp, W_self1, W_neigh2, b1.reshape(1, feat))

    qa, qb = agg(y2a, y2b, src3d, dst3d)

    out = pl.pallas_call(
        _fin_body, grid=grid,
        in_specs=[row_blk, part_blk, part_blk, deg_blk,
                  w_blk, b_blk,
                  pl.BlockSpec((feat, nclass), lambda i: (0, 0)),
                  pl.BlockSpec((1, nclass), lambda i: (0, 0))],
        out_specs=pl.BlockSpec((BR, nclass), lambda i: (i, 0)),
        out_shape=jax.ShapeDtypeStruct((n_nodes, nclass), F32),
    )(h1, qa, qb, degp, W_self2, b2.reshape(1, feat),
      Wfc, bfc.reshape(1, nclass))

    return out


# speedup vs baseline: 1.0133x; 1.0133x over previous
import jax
import jax.numpy as jnp
from jax import lax
from jax.experimental import pallas as pl
from jax.experimental.pallas import tpu as pltpu
from jax.experimental.pallas import tpu_sc as plsc

F32 = jnp.float32

NC = 2
NS = 16
NW = NC * NS

CH = 125
DEGW = 16
ZR = 64
NBUF = 4


def _agg_kernel(n_pad, fh, n_ch):
    rows_per_tile = n_pad // NS
    n_zb = rows_per_tile // ZR

    mesh = plsc.VectorSubcoreMesh(
        core_axis_name="c", subcore_axis_name="s",
        num_cores=NC, num_subcores=NS)

    out_type = (jax.ShapeDtypeStruct((NC, n_pad, fh), F32),
                jax.ShapeDtypeStruct((NC, n_pad, fh), F32))
    scratch = (
        [pltpu.VMEM((n_ch, CH), jnp.int32),
         pltpu.VMEM((n_ch, CH), jnp.int32)]
        + [pltpu.VMEM((CH, fh), F32)] * NBUF
        + [pltpu.VMEM((ZR, fh), F32),
           pltpu.VMEM((ZR, fh), F32),
           pltpu.VMEM_SHARED((n_pad, fh), F32)]
        + [pltpu.SemaphoreType.DMA] * NBUF
    )

    def body(ya_hbm, yb_hbm, src_hbm, dst_hbm, outa_hbm, outb_hbm, *rest):
        srcv, dstv = rest[0], rest[1]
        rows = rest[2:2 + NBUF]
        zb, stg, acc_sh = rest[2 + NBUF:5 + NBUF]
        sems = rest[5 + NBUF:5 + 2 * NBUF]
        c = lax.axis_index("c")
        s = lax.axis_index("s")
        wid = c * NS + s

        def zrow(r, _):
            for k in range(fh // 16):
                zb[r, pl.ds(k * 16, 16)] = jnp.zeros((16,), F32)
            return 0
        lax.fori_loop(0, ZR, zrow, 0)
        for j in range(n_zb):
            pltpu.sync_copy(zb, acc_sh.at[pl.ds(s * rows_per_tile + j * ZR, ZR)])

        pltpu.sync_copy(src_hbm.at[wid], srcv)
        pltpu.sync_copy(dst_hbm.at[wid], dstv)

        plsc.subcore_barrier()

        for y_hbm, out_hbm, first in ((ya_hbm, outa_hbm, True),
                                      (yb_hbm, outb_hbm, False)):
            for b in range(NBUF):
                pltpu.async_copy(y_hbm.at[srcv.at[b]], rows[b], sems[b])

            def step(i, _):
                j0 = NBUF * i
                for b in range(NBUF):
                    j = j0 + b
                    pltpu.make_async_copy(
                        y_hbm.at[srcv.at[j]], rows[b], sems[b]).wait()
                    pltpu.sync_copy(rows[b], acc_sh.at[dstv.at[j]],
                                    add=True)

                    @pl.when(j + NBUF < n_ch)
                    def _(b=b, j=j):
                        pltpu.async_copy(
                            y_hbm.at[srcv.at[j + NBUF]], rows[b], sems[b])
                return 0
            lax.fori_loop(0, n_ch // NBUF, step, 0)

            plsc.subcore_barrier()

            for j in range(n_zb):
                r0 = s * rows_per_tile + j * ZR
                pltpu.sync_copy(acc_sh.at[pl.ds(r0, ZR)], stg)
                pltpu.sync_copy(stg, out_hbm.at[c].at[pl.ds(r0, ZR)])
                if first:
                    pltpu.sync_copy(zb, acc_sh.at[pl.ds(r0, ZR)])

            if first:
                plsc.subcore_barrier()

    return pl.kernel(body, out_type=out_type, mesh=mesh,
                     scratch_types=list(scratch),
                     compiler_params=pltpu.CompilerParams(
                         use_tc_tiling_on_sc=False))


def _deg_kernel(n_pad, n_ch):
    rows_per_tile = n_pad // NS

    mesh = plsc.VectorSubcoreMesh(
        core_axis_name="c", subcore_axis_name="s",
        num_cores=NC, num_subcores=NS)

    out_type = jax.ShapeDtypeStruct((NC, n_pad, DEGW), F32)
    scratch = [
        pltpu.VMEM((n_ch, CH), jnp.int32),
        pltpu.VMEM((CH, DEGW), F32),
        pltpu.VMEM((rows_per_tile, DEGW), F32),
        pltpu.VMEM_SHARED((n_pad, DEGW), F32),
    ]

    def body(dst_hbm, deg_hbm, dstv, ones_v, degz, deg_sh):
        c = lax.axis_index("c")
        s = lax.axis_index("s")
        wid = c * NS + s

        def zdrow(r, _):
            degz[r] = jnp.zeros((DEGW,), F32)
            return 0
        lax.fori_loop(0, rows_per_tile, zdrow, 0)
        pltpu.sync_copy(degz, deg_sh.at[pl.ds(s * rows_per_tile, rows_per_tile)])

        def orow(r, _):
            ones_v[r] = jnp.ones((DEGW,), F32)
            return 0
        lax.fori_loop(0, CH, orow, 0)

        pltpu.sync_copy(dst_hbm.at[wid], dstv)

        plsc.subcore_barrier()

        def chunk(j, _):
            pltpu.sync_copy(ones_v, deg_sh.at[dstv.at[j]], add=True)
            return 0
        lax.fori_loop(0, n_ch, chunk, 0)

        plsc.subcore_barrier()

        r0 = s * rows_per_tile
        pltpu.sync_copy(deg_sh.at[pl.ds(r0, rows_per_tile)], degz)
        pltpu.sync_copy(degz, deg_hbm.at[c].at[pl.ds(r0, rows_per_tile)])

    return pl.kernel(body, out_type=out_type, mesh=mesh,
                     scratch_types=scratch,
                     compiler_params=pltpu.CompilerParams(
                         use_tc_tiling_on_sc=False))


def _mm1_body(x_ref, w_ref, oa_ref, ob_ref):
    fh = oa_ref.shape[1]
    y = jnp.dot(x_ref[...], w_ref[...], preferred_element_type=F32)
    oa_ref[...] = y[:, :fh]
    ob_ref[...] = y[:, fh:]


def _mean(pa, pb, d):
    deg = d[0][:, :1] + d[1][:, :1]
    rd = 1.0 / jnp.maximum(deg, 1.0)
    return jnp.concatenate([(pa[0] + pa[1]) * rd, (pb[0] + pb[1]) * rd],
                           axis=1)


def _mid_body(x_ref, pa_ref, pb_ref, d_ref, ws_ref, wn_ref, b_ref,
              h_ref, ya_ref, yb_ref):
    fh = ya_ref.shape[1]
    mean = _mean(pa_ref[...], pb_ref[...], d_ref[...])
    h = jnp.maximum(
        jnp.dot(x_ref[...], ws_ref[...], preferred_element_type=F32)
        + mean + b_ref[...], 0.0)
    h_ref[...] = h
    y = jnp.dot(h, wn_ref[...], preferred_element_type=F32)
    ya_ref[...] = y[:, :fh]
    yb_ref[...] = y[:, fh:]


def _fin_body(h_ref, pa_ref, pb_ref, d_ref, ws_ref, b_ref,
              wfc_ref, bfc_ref, o_ref):
    mean = _mean(pa_ref[...], pb_ref[...], d_ref[...])
    h2 = (jnp.dot(h_ref[...], ws_ref[...], preferred_element_type=F32)
          + mean + b_ref[...])
    o_ref[...] = (jnp.dot(h2, wfc_ref[...], preferred_element_type=F32)
                  + bfc_ref[...])


def kernel(x, edge_index, W_self1, W_neigh1, b1, W_self2, W_neigh2, b2,
           Wfc, bfc):
    n_nodes, feat = x.shape
    n_edges = edge_index.shape[1]
    nclass = Wfc.shape[1]
    fh = feat // 2
    n_pad = -(-n_nodes // (NS * 128)) * (NS * 128)

    e_per_tile = n_edges // NW
    n_ch = e_per_tile // CH
    src3d = edge_index[0].reshape(NW, n_ch, CH)
    dst3d = edge_index[1].reshape(NW, n_ch, CH)

    agg = _agg_kernel(n_pad, fh, n_ch)
    deg_k = _deg_kernel(n_pad, n_ch)

    BR = 1000
    grid = (n_nodes // BR,)
    row_blk = pl.BlockSpec((BR, feat), lambda i: (i, 0))
    half_blk = pl.BlockSpec((BR, fh), lambda i: (i, 0))
    part_blk = pl.BlockSpec((NC, BR, fh), lambda i: (0, i, 0))
    deg_blk = pl.BlockSpec((NC, BR, DEGW), lambda i: (0, i, 0))
    w_blk = pl.BlockSpec((feat, feat), lambda i: (0, 0))
    b_blk = pl.BlockSpec((1, feat), lambda i: (0, 0))
    half_shape = jax.ShapeDtypeStruct((n_nodes, fh), F32)

    y1a, y1b = pl.pallas_call(
        _mm1_body, grid=grid,
        in_specs=[row_blk, w_blk],
        out_specs=(half_blk, half_blk),
        out_shape=(half_shape, half_shape),
    )(x, W_neigh1)

    degp = deg_k(dst3d)
    pa, pb = agg(y1a, y1b, src3d, dst3d)

    h1, y2a, y2b = pl.pallas_call(
        _mid_body, grid=grid,
        in_specs=[row_blk, part_blk, part_blk, deg_blk,
                  w_blk, w_blk, b_blk],
        out_specs=(row_blk, half_blk, half_blk),
        out_shape=(jax.ShapeDtypeStruct((n_nodes, feat), F32),
                   half_shape, half_shape),
    )(x, pa, pb, degp, W_self1, W_neigh2, b1.reshape(1, feat))

    qa, qb = agg(y2a, y2b, src3d, dst3d)

    out = pl.pallas_call(
        _fin_body, grid=grid,
        in_specs=[row_blk, part_blk, part_blk, deg_blk,
                  w_blk, b_blk,
                  pl.BlockSpec((feat, nclass), lambda i: (0, 0)),
                  pl.BlockSpec((1, nclass), lambda i: (0, 0))],
        out_specs=pl.BlockSpec((BR, nclass), lambda i: (i, 0)),
        out_shape=jax.ShapeDtypeStruct((n_nodes, nclass), F32),
    )(h1, qa, qb, degp, W_self2, b2.reshape(1, feat),
      Wfc, bfc.reshape(1, nclass))

    return out


# speedup vs baseline: 1.0285x; 1.0150x over previous
import jax
import jax.numpy as jnp
from jax import lax
from jax.experimental import pallas as pl
from jax.experimental.pallas import tpu as pltpu
from jax.experimental.pallas import tpu_sc as plsc

F32 = jnp.float32

NC = 2
NS = 16
NW = NC * NS

CH = 125
DEGW = 16
ZR = 128
NBUF = 4


def _agg_kernel(n_pad, fh, n_ch):
    rows_per_tile = n_pad // NS
    n_zb = rows_per_tile // ZR

    mesh = plsc.VectorSubcoreMesh(
        core_axis_name="c", subcore_axis_name="s",
        num_cores=NC, num_subcores=NS)

    out_type = (jax.ShapeDtypeStruct((NC, n_pad, fh), F32),
                jax.ShapeDtypeStruct((NC, n_pad, fh), F32))
    scratch = (
        [pltpu.VMEM((n_ch, CH), jnp.int32),
         pltpu.VMEM((n_ch, CH), jnp.int32)]
        + [pltpu.VMEM((CH, fh), F32)] * NBUF
        + [pltpu.VMEM((ZR, fh), F32),
           pltpu.VMEM((ZR, fh), F32),
           pltpu.VMEM_SHARED((n_pad, fh), F32)]
        + [pltpu.SemaphoreType.DMA] * NBUF
    )

    def body(ya_hbm, yb_hbm, src_hbm, dst_hbm, outa_hbm, outb_hbm, *rest):
        srcv, dstv = rest[0], rest[1]
        rows = rest[2:2 + NBUF]
        zb, stg, acc_sh = rest[2 + NBUF:5 + NBUF]
        sems = rest[5 + NBUF:5 + 2 * NBUF]
        c = lax.axis_index("c")
        s = lax.axis_index("s")
        wid = c * NS + s

        def zrow(r, _):
            for k in range(fh // 16):
                zb[r, pl.ds(k * 16, 16)] = jnp.zeros((16,), F32)
            return 0
        lax.fori_loop(0, ZR, zrow, 0)
        for j in range(n_zb):
            pltpu.sync_copy(zb, acc_sh.at[pl.ds(s * rows_per_tile + j * ZR, ZR)])

        pltpu.sync_copy(src_hbm.at[wid], srcv)
        pltpu.sync_copy(dst_hbm.at[wid], dstv)

        plsc.subcore_barrier()

        for y_hbm, out_hbm, first in ((ya_hbm, outa_hbm, True),
                                      (yb_hbm, outb_hbm, False)):
            for b in range(NBUF):
                pltpu.async_copy(y_hbm.at[srcv.at[b]], rows[b], sems[b])

            def step(i, _):
                j0 = NBUF * i
                for b in range(NBUF):
                    j = j0 + b
                    pltpu.make_async_copy(
                        y_hbm.at[srcv.at[j]], rows[b], sems[b]).wait()
                    pltpu.sync_copy(rows[b], acc_sh.at[dstv.at[j]],
                                    add=True)

                    @pl.when(j + NBUF < n_ch)
                    def _(b=b, j=j):
                        pltpu.async_copy(
                            y_hbm.at[srcv.at[j + NBUF]], rows[b], sems[b])
                return 0
            lax.fori_loop(0, n_ch // NBUF, step, 0)

            plsc.subcore_barrier()

            for j in range(n_zb):
                r0 = s * rows_per_tile + j * ZR
                pltpu.sync_copy(acc_sh.at[pl.ds(r0, ZR)], stg)
                pltpu.sync_copy(stg, out_hbm.at[c].at[pl.ds(r0, ZR)])
                if first:
                    pltpu.sync_copy(zb, acc_sh.at[pl.ds(r0, ZR)])

            if first:
                plsc.subcore_barrier()

    return pl.kernel(body, out_type=out_type, mesh=mesh,
                     scratch_types=list(scratch),
                     compiler_params=pltpu.CompilerParams(
                         use_tc_tiling_on_sc=False))


def _deg_kernel(n_pad, n_ch):
    rows_per_tile = n_pad // NS

    mesh = plsc.VectorSubcoreMesh(
        core_axis_name="c", subcore_axis_name="s",
        num_cores=NC, num_subcores=NS)

    out_type = jax.ShapeDtypeStruct((NC, n_pad, DEGW), F32)
    scratch = [
        pltpu.VMEM((n_ch, CH), jnp.int32),
        pltpu.VMEM((CH, DEGW), F32),
        pltpu.VMEM((rows_per_tile, DEGW), F32),
        pltpu.VMEM_SHARED((n_pad, DEGW), F32),
    ]

    def body(dst_hbm, deg_hbm, dstv, ones_v, degz, deg_sh):
        c = lax.axis_index("c")
        s = lax.axis_index("s")
        wid = c * NS + s

        def zdrow(r, _):
            degz[r] = jnp.zeros((DEGW,), F32)
            return 0
        lax.fori_loop(0, rows_per_tile, zdrow, 0)
        pltpu.sync_copy(degz, deg_sh.at[pl.ds(s * rows_per_tile, rows_per_tile)])

        def orow(r, _):
            ones_v[r] = jnp.ones((DEGW,), F32)
            return 0
        lax.fori_loop(0, CH, orow, 0)

        pltpu.sync_copy(dst_hbm.at[wid], dstv)

        plsc.subcore_barrier()

        def chunk(j, _):
            pltpu.sync_copy(ones_v, deg_sh.at[dstv.at[j]], add=True)
            return 0
        lax.fori_loop(0, n_ch, chunk, 0)

        plsc.subcore_barrier()

        r0 = s * rows_per_tile
        pltpu.sync_copy(deg_sh.at[pl.ds(r0, rows_per_tile)], degz)
        pltpu.sync_copy(degz, deg_hbm.at[c].at[pl.ds(r0, rows_per_tile)])

    return pl.kernel(body, out_type=out_type, mesh=mesh,
                     scratch_types=scratch,
                     compiler_params=pltpu.CompilerParams(
                         use_tc_tiling_on_sc=False))


def _mm1_body(x_ref, w_ref, oa_ref, ob_ref):
    fh = oa_ref.shape[1]
    y = jnp.dot(x_ref[...], w_ref[...], preferred_element_type=F32)
    oa_ref[...] = y[:, :fh]
    ob_ref[...] = y[:, fh:]


def _mean(pa, pb, d):
    deg = d[0][:, :1] + d[1][:, :1]
    rd = 1.0 / jnp.maximum(deg, 1.0)
    return jnp.concatenate([(pa[0] + pa[1]) * rd, (pb[0] + pb[1]) * rd],
                           axis=1)


def _mid_body(x_ref, pa_ref, pb_ref, d_ref, ws_ref, wn_ref, b_ref,
              h_ref, ya_ref, yb_ref):
    fh = ya_ref.shape[1]
    mean = _mean(pa_ref[...], pb_ref[...], d_ref[...])
    h = jnp.maximum(
        jnp.dot(x_ref[...], ws_ref[...], preferred_element_type=F32)
        + mean + b_ref[...], 0.0)
    h_ref[...] = h
    y = jnp.dot(h, wn_ref[...], preferred_element_type=F32)
    ya_ref[...] = y[:, :fh]
    yb_ref[...] = y[:, fh:]


def _fin_body(h_ref, pa_ref, pb_ref, d_ref, ws_ref, b_ref,
              wfc_ref, bfc_ref, o_ref):
    mean = _mean(pa_ref[...], pb_ref[...], d_ref[...])
    h2 = (jnp.dot(h_ref[...], ws_ref[...], preferred_element_type=F32)
          + mean + b_ref[...])
    o_ref[...] = (jnp.dot(h2, wfc_ref[...], preferred_element_type=F32)
                  + bfc_ref[...])


def kernel(x, edge_index, W_self1, W_neigh1, b1, W_self2, W_neigh2, b2,
           Wfc, bfc):
    n_nodes, feat = x.shape
    n_edges = edge_index.shape[1]
    nclass = Wfc.shape[1]
    fh = feat // 2
    n_pad = -(-n_nodes // (NS * 128)) * (NS * 128)

    e_per_tile = n_edges // NW
    n_ch = e_per_tile // CH
    src3d = edge_index[0].reshape(NW, n_ch, CH)
    dst3d = edge_index[1].reshape(NW, n_ch, CH)

    agg = _agg_kernel(n_pad, fh, n_ch)
    deg_k = _deg_kernel(n_pad, n_ch)

    BR = 1000
    grid = (n_nodes // BR,)
    row_blk = pl.BlockSpec((BR, feat), lambda i: (i, 0))
    half_blk = pl.BlockSpec((BR, fh), lambda i: (i, 0))
    part_blk = pl.BlockSpec((NC, BR, fh), lambda i: (0, i, 0))
    deg_blk = pl.BlockSpec((NC, BR, DEGW), lambda i: (0, i, 0))
    w_blk = pl.BlockSpec((feat, feat), lambda i: (0, 0))
    b_blk = pl.BlockSpec((1, feat), lambda i: (0, 0))
    half_shape = jax.ShapeDtypeStruct((n_nodes, fh), F32)

    y1a, y1b = pl.pallas_call(
        _mm1_body, grid=grid,
        in_specs=[row_blk, w_blk],
        out_specs=(half_blk, half_blk),
        out_shape=(half_shape, half_shape),
    )(x, W_neigh1)

    degp = deg_k(dst3d)
    pa, pb = agg(y1a, y1b, src3d, dst3d)

    h1, y2a, y2b = pl.pallas_call(
        _mid_body, grid=grid,
        in_specs=[row_blk, part_blk, part_blk, deg_blk,
                  w_blk, w_blk, b_blk],
        out_specs=(row_blk, half_blk, half_blk),
        out_shape=(jax.ShapeDtypeStruct((n_nodes, feat), F32),
                   half_shape, half_shape),
    )(x, pa, pb, degp, W_self1, W_neigh2, b1.reshape(1, feat))

    qa, qb = agg(y2a, y2b, src3d, dst3d)

    out = pl.pallas_call(
        _fin_body, grid=grid,
        in_specs=[row_blk, part_blk, part_blk, deg_blk,
                  w_blk, b_blk,
                  pl.BlockSpec((feat, nclass), lambda i: (0, 0)),
                  pl.BlockSpec((1, nclass), lambda i: (0, 0))],
        out_specs=pl.BlockSpec((BR, nclass), lambda i: (i, 0)),
        out_shape=jax.ShapeDtypeStruct((n_nodes, nclass), F32),
    )(h1, qa, qb, degp, W_self2, b2.reshape(1, feat),
      Wfc, bfc.reshape(1, nclass))

    return out
